# Initial kernel scaffold; baseline (speedup 1.0000x reference)
import jax
import jax.numpy as jnp
from jax.experimental import pallas as pl


def kernel(x, edge_index, batch, W1, b1, W2, b2, gamma, beta, Wf, bf):
    raise NotImplementedError("write your pallas kernel here")


# speedup vs baseline: 1.0028x; 1.0028x over previous
import jax
import jax.numpy as jnp
from jax.experimental import pallas as pl

N_ = 10000
D_ = 128
H_ = 128
OUT_ = 64
L_ = 5
G_ = 128
EPS_ = 1e-5


def _dot_body(a_ref, b_ref, o_ref):
    o_ref[...] = jnp.dot(a_ref[...], b_ref[...],
                         preferred_element_type=jnp.float32)


_dot_call = pl.pallas_call(
    _dot_body, out_shape=jax.ShapeDtypeStruct((N_, H_), jnp.float32))


def _rne16f(x):
    u = jax.lax.bitcast_convert_type(x, jnp.uint32)
    u = (u + jnp.uint32(0x7FFF) + ((u >> 16) & jnp.uint32(1))) & jnp.uint32(
        0xFFFF0000)
    return jax.lax.bitcast_convert_type(u, jnp.float32)


def _pool_final_body(h_ref, batch_ref, wf_ref, bf_ref, o_ref):
    gid = jax.lax.broadcasted_iota(jnp.int32, (1, G_), 1)
    onehot = jnp.where(batch_ref[...] == gid, 1.0, 0.0)
    h = h_ref[...]
    h1 = _rne16f(h)
    r = h - h1
    h2 = _rne16f(r)
    h3 = r - h2
    dg = lambda a, b: jax.lax.dot_general(
        a, b, (((0,), (0,)), ((), ())), preferred_element_type=jnp.float32)
    pooled = dg(onehot, h1) + dg(onehot, h2) + dg(onehot, h3)
    o_ref[...] = jnp.maximum(
        jnp.dot(pooled, wf_ref[...], preferred_element_type=jnp.float32)
        + bf_ref[...], 0.0)


_pool_final_call = pl.pallas_call(
    _pool_final_body, out_shape=jax.ShapeDtypeStruct((G_, OUT_), jnp.float32))


def kernel(x, edge_index, batch, W1, b1, W2, b2, gamma, beta, Wf, bf):
    src = edge_index[0]
    dst = edge_index[1]
    h = x
    for i in range(L_):
        msg = jnp.take(h, src, axis=0)
        agg = jax.ops.segment_sum(msg, dst, num_segments=N_)
        z = agg + h
        z = jnp.maximum(_dot_call(z, W1[i]) + b1[i], 0.0)
        z = jnp.maximum(z @ W2[i] + b2[i], 0.0)
        mean = jnp.mean(z, axis=0)
        var = jnp.var(z, axis=0)
        h = (z - mean) / jnp.sqrt(var + EPS_) * gamma[i] + beta[i]
    return _pool_final_call(h, batch.reshape(N_, 1), Wf, bf.reshape(1, OUT_))
